# Initial kernel scaffold; baseline (speedup 1.0000x reference)
import jax
import jax.numpy as jnp
from jax.experimental import pallas as pl


def kernel(x, edge_index, W1, b1, W2, b2, CW1, Cb1, CW2, Cb2):
    raise NotImplementedError("write your pallas kernel here")


# speedup vs baseline: 30.7502x; 30.7502x over previous
import functools

import jax
import jax.numpy as jnp
from jax import lax
from jax.experimental import pallas as pl
from jax.experimental.pallas import tpu as pltpu
from jax.experimental.pallas import tpu_sc as plsc

N = 100000
E = 3200000
F = 16
NC, NS = 2, 16
NW = NC * NS
C = 128
K = 784
S = 16
NSB = K // S
E_PAD = NW * K * C
RPT = 6272
NPAD = NS * RPT
RB = 1024
GRID = NPAD // RB


def _widx():
    return lax.axis_index("c") * NS + lax.axis_index("s")


def _sc_mesh():
    return plsc.VectorSubcoreMesh(core_axis_name="c", subcore_axis_name="s")


def _deg_body(dst_i, zeros_hbm, out_hbm, acc_sh, ones_v, dst_v):
    c = lax.axis_index("c")
    s = lax.axis_index("s")
    wid = c * NS + s
    r0 = s * RPT
    def fill(i, carry):
        ones_v[i] = jnp.ones((F,), jnp.float32)
        return carry
    lax.fori_loop(0, C, fill, 0)
    pltpu.sync_copy(zeros_hbm.at[pl.ds(r0, RPT)], acc_sh.at[pl.ds(r0, RPT)])
    plsc.subcore_barrier()

    def sb_body(sb, carry):
        pltpu.sync_copy(dst_i.at[wid, pl.ds(sb * S, S)], dst_v)
        def ch(j, cc):
            pltpu.sync_copy(ones_v, acc_sh.at[dst_v.at[j]], add=True)
            return cc
        return lax.fori_loop(0, S, ch, carry)
    lax.fori_loop(0, NSB, sb_body, 0)

    plsc.subcore_barrier()
    pltpu.sync_copy(acc_sh.at[pl.ds(r0, RPT)], out_hbm.at[c, pl.ds(r0, RPT)])


_sc_deg = pl.kernel(
    _deg_body,
    out_type=jax.ShapeDtypeStruct((NC, NPAD, F), jnp.float32),
    mesh=_sc_mesh(),
    compiler_params=pltpu.CompilerParams(use_tc_tiling_on_sc=False),
    scratch_types=[
        pltpu.VMEM_SHARED((NPAD, F), jnp.float32),
        pltpu.VMEM((C, F), jnp.float32),
        pltpu.VMEM((S, C), jnp.int32),
    ],
)


def _agg_body(g_hbm, src_i, dst_i, zeros_hbm, out_hbm,
              acc_sh, src_v, dst_v, rows_v, sem):
    c = lax.axis_index("c")
    s = lax.axis_index("s")
    wid = c * NS + s
    r0 = s * RPT
    pltpu.sync_copy(zeros_hbm.at[pl.ds(r0, RPT)], acc_sh.at[pl.ds(r0, RPT)])
    plsc.subcore_barrier()

    def sb_body(sb, carry):
        pltpu.sync_copy(src_i.at[wid, pl.ds(sb * S, S)], src_v)
        pltpu.sync_copy(dst_i.at[wid, pl.ds(sb * S, S)], dst_v)
        def ch(j, cc):
            pltpu.async_copy(g_hbm.at[src_v.at[j]], rows_v, sem).wait()
            pltpu.sync_copy(rows_v, acc_sh.at[dst_v.at[j]], add=True)
            return cc
        return lax.fori_loop(0, S, ch, carry)
    lax.fori_loop(0, NSB, sb_body, 0)

    plsc.subcore_barrier()
    pltpu.sync_copy(acc_sh.at[pl.ds(r0, RPT)], out_hbm.at[c, pl.ds(r0, RPT)])


_sc_agg = pl.kernel(
    _agg_body,
    out_type=jax.ShapeDtypeStruct((NC, NPAD, F), jnp.float32),
    mesh=_sc_mesh(),
    compiler_params=pltpu.CompilerParams(use_tc_tiling_on_sc=False),
    scratch_types=[
        pltpu.VMEM_SHARED((NPAD, F), jnp.float32),
        pltpu.VMEM((S, C), jnp.int32),
        pltpu.VMEM((S, C), jnp.int32),
        pltpu.VMEM((C, F), jnp.float32),
        pltpu.SemaphoreType.DMA,
    ],
)


def _row_mask(shape):
    i = pl.program_id(0)
    row = lax.broadcasted_iota(jnp.int32, shape, 0) + i * RB
    return row < N


def _tc_a_body(x_ref, d0_ref, d1_ref, w1_ref, dinv_ref, g1_ref):
    dinv = lax.rsqrt(d0_ref[...] + d1_ref[...] + 1.0)
    h = jnp.dot(x_ref[...], w1_ref[...], preferred_element_type=jnp.float32)
    dinv_ref[...] = dinv
    g1_ref[...] = jnp.where(_row_mask((RB, F)), h * dinv, 0.0)


def _tc_b_body(a0_ref, a1_ref, g_ref, dinv_ref, b_ref, w2_ref, g2_ref):
    dinv = dinv_ref[...]
    h = dinv * (a0_ref[...] + a1_ref[...] + g_ref[...]) + b_ref[...]
    h = jnp.maximum(h, 0.0)
    h2 = jnp.dot(h, w2_ref[...], preferred_element_type=jnp.float32)
    g2_ref[...] = jnp.where(_row_mask((RB, F)), h2 * dinv, 0.0)


def _tc_c_body(a0_ref, a1_ref, g_ref, dinv_ref, b_ref,
               cw1_ref, cb1_ref, cw2_ref, cb2_ref, o1_ref, o2_ref):
    h = dinv_ref[...] * (a0_ref[...] + a1_ref[...] + g_ref[...]) + b_ref[...]
    h = jnp.maximum(h, 0.0)
    o1_ref[...] = jnp.dot(h, cw1_ref[...],
                          preferred_element_type=jnp.float32) + cb1_ref[...]
    o2_ref[...] = jnp.dot(h, cw2_ref[...],
                          preferred_element_type=jnp.float32) + cb2_ref[...]


def _rows(width):
    return pl.BlockSpec((RB, width), lambda i: (i, 0))


def _full(shape):
    return pl.BlockSpec(shape, lambda i: tuple(0 for _ in shape))


_tc_a = pl.pallas_call(
    _tc_a_body,
    grid=(GRID,),
    in_specs=[_rows(6), _rows(F), _rows(F), _full((6, F))],
    out_specs=[_rows(F), _rows(F)],
    out_shape=[jax.ShapeDtypeStruct((NPAD, F), jnp.float32),
               jax.ShapeDtypeStruct((NPAD, F), jnp.float32)],
)

_tc_b = pl.pallas_call(
    _tc_b_body,
    grid=(GRID,),
    in_specs=[_rows(F), _rows(F), _rows(F), _rows(F),
              _full((1, F)), _full((F, F))],
    out_specs=[_rows(F)],
    out_shape=[jax.ShapeDtypeStruct((NPAD, F), jnp.float32)],
)

_tc_c = pl.pallas_call(
    _tc_c_body,
    grid=(GRID,),
    in_specs=[_rows(F), _rows(F), _rows(F), _rows(F), _full((1, F)),
              _full((F, 13)), _full((1, 13)), _full((F, 8)), _full((1, 8))],
    out_specs=[_rows(13), _rows(8)],
    out_shape=[jax.ShapeDtypeStruct((NPAD, 13), jnp.float32),
               jax.ShapeDtypeStruct((NPAD, 8), jnp.float32)],
)


@jax.jit
def kernel(x, edge_index, W1, b1, W2, b2, CW1, Cb1, CW2, Cb2):
    src = edge_index[0].astype(jnp.int32)
    dst = edge_index[1].astype(jnp.int32)
    src_i = jnp.full((E_PAD,), N, jnp.int32).at[:E].set(src).reshape(NW, K, C)
    dst_i = jnp.full((E_PAD,), N, jnp.int32).at[:E].set(dst).reshape(NW, K, C)
    xp = jnp.zeros((NPAD, 6), jnp.float32).at[:N].set(x)
    zeros16 = jnp.zeros((NPAD, F), jnp.float32)

    deg = _sc_deg(dst_i, zeros16)
    dinv, g1 = _tc_a(xp, deg[0], deg[1], W1)
    agg1 = _sc_agg(g1, src_i, dst_i, zeros16)
    (g2,) = _tc_b(agg1[0], agg1[1], g1, dinv, b1.reshape(1, F), W2)
    agg2 = _sc_agg(g2, src_i, dst_i, zeros16)
    o1, o2 = _tc_c(agg2[0], agg2[1], g2, dinv, b2.reshape(1, F),
                   CW1, Cb1.reshape(1, 13), CW2, Cb2.reshape(1, 8))
    return o1[:N], o2[:N]


# speedup vs baseline: 50.7819x; 1.6514x over previous
import jax
import jax.numpy as jnp
from jax import lax
from jax.experimental import pallas as pl
from jax.experimental.pallas import tpu as pltpu
from jax.experimental.pallas import tpu_sc as plsc

N = 100000
E = 3200000
F = 16
NC, NS = 2, 16
NW = NC * NS
C = 128
K = 784
S = 4
NSB = K // S
E_PAD = NW * K * C
RPT = 6400
NPAD = NS * RPT
RB = 4096
GRID = NPAD // RB
SD = 8
NSBD = K // SD


def _sc_mesh():
    return plsc.VectorSubcoreMesh(core_axis_name="c", subcore_axis_name="s")


def _deg_body(dst_i, zeros_hbm, out_hbm, acc_sh, ones_v, dst_v, isem):
    c = lax.axis_index("c")
    s = lax.axis_index("s")
    wid = c * NS + s
    r0 = s * RPT

    def fill(i, carry):
        ones_v[i] = jnp.ones((F,), jnp.float32)
        return carry
    lax.fori_loop(0, C, fill, 0)
    pltpu.sync_copy(zeros_hbm.at[pl.ds(r0, RPT)], acc_sh.at[pl.ds(r0, RPT)])
    plsc.subcore_barrier()

    def stage(sb, slot):
        pltpu.async_copy(dst_i.at[wid, pl.ds(sb * SD, SD)], dst_v.at[slot],
                         isem.at[slot])

    def wait_stage(sb, slot):
        pltpu.make_async_copy(dst_i.at[wid, pl.ds(sb * SD, SD)],
                              dst_v.at[slot], isem.at[slot]).wait()

    stage(0, 0)
    stage(1, 1)

    def sb_body(sb, carry):
        slot = lax.rem(sb, 2)
        wait_stage(sb, slot)

        def ch(j, cc):
            pltpu.sync_copy(ones_v, acc_sh.at[dst_v.at[slot, j]], add=True)
            return cc
        lax.fori_loop(0, SD, ch, 0)

        @pl.when(sb + 2 < NSBD)
        def _():
            stage(sb + 2, slot)
        return carry
    lax.fori_loop(0, NSBD, sb_body, 0)

    plsc.subcore_barrier()
    pltpu.sync_copy(acc_sh.at[pl.ds(r0, RPT)], out_hbm.at[c, pl.ds(r0, RPT)])


_sc_deg = pl.kernel(
    _deg_body,
    out_type=jax.ShapeDtypeStruct((NC, NPAD, F), jnp.float32),
    mesh=_sc_mesh(),
    compiler_params=pltpu.CompilerParams(use_tc_tiling_on_sc=False),
    scratch_types=[
        pltpu.VMEM_SHARED((NPAD, F), jnp.float32),
        pltpu.VMEM((C, F), jnp.float32),
        pltpu.VMEM((2, SD, C), jnp.int32),
        pltpu.SemaphoreType.DMA((2,)),
    ],
)


def _agg_body(g_hbm, src_i, dst_i, zeros_hbm, out_hbm,
              acc_sh, src_v, dst_v, rows_v, isem, gsem):
    c = lax.axis_index("c")
    s = lax.axis_index("s")
    wid = c * NS + s
    r0 = s * RPT
    pltpu.sync_copy(zeros_hbm.at[pl.ds(r0, RPT)], acc_sh.at[pl.ds(r0, RPT)])
    plsc.subcore_barrier()

    def stage(sb, slot):
        pltpu.async_copy(src_i.at[wid, pl.ds(sb * S, S)], src_v.at[slot],
                         isem.at[slot, 0])
        pltpu.async_copy(dst_i.at[wid, pl.ds(sb * S, S)], dst_v.at[slot],
                         isem.at[slot, 1])

    def wait_stage(sb, slot):
        pltpu.make_async_copy(src_i.at[wid, pl.ds(sb * S, S)],
                              src_v.at[slot], isem.at[slot, 0]).wait()
        pltpu.make_async_copy(dst_i.at[wid, pl.ds(sb * S, S)],
                              dst_v.at[slot], isem.at[slot, 1]).wait()

    def fire(slot):
        def f(j, carry):
            pltpu.async_copy(g_hbm.at[src_v.at[slot, j]], rows_v.at[slot, j],
                             gsem.at[slot, j])
            return carry
        lax.fori_loop(0, S, f, 0)

    def drain(slot):
        def f(j, carry):
            pltpu.make_async_copy(g_hbm.at[src_v.at[slot, j]],
                                  rows_v.at[slot, j], gsem.at[slot, j]).wait()
            pltpu.sync_copy(rows_v.at[slot, j], acc_sh.at[dst_v.at[slot, j]],
                            add=True)
            return carry
        lax.fori_loop(0, S, f, 0)

    stage(0, 0)
    stage(1, 1)
    wait_stage(0, 0)
    fire(0)

    def sb_body(sb, carry):
        slot = lax.rem(sb, 2)
        nslot = 1 - slot

        @pl.when(sb + 1 < NSB)
        def _():
            wait_stage(sb + 1, nslot)
            fire(nslot)
        drain(slot)

        @pl.when(sb + 2 < NSB)
        def _():
            stage(sb + 2, slot)
        return carry
    lax.fori_loop(0, NSB, sb_body, 0)

    plsc.subcore_barrier()
    pltpu.sync_copy(acc_sh.at[pl.ds(r0, RPT)], out_hbm.at[c, pl.ds(r0, RPT)])


_sc_agg = pl.kernel(
    _agg_body,
    out_type=jax.ShapeDtypeStruct((NC, NPAD, F), jnp.float32),
    mesh=_sc_mesh(),
    compiler_params=pltpu.CompilerParams(use_tc_tiling_on_sc=False),
    scratch_types=[
        pltpu.VMEM_SHARED((NPAD, F), jnp.float32),
        pltpu.VMEM((2, S, C), jnp.int32),
        pltpu.VMEM((2, S, C), jnp.int32),
        pltpu.VMEM((2, S, C, F), jnp.float32),
        pltpu.SemaphoreType.DMA((2, 2)),
        pltpu.SemaphoreType.DMA((2, S)),
    ],
)


def _row_mask(shape):
    i = pl.program_id(0)
    row = lax.broadcasted_iota(jnp.int32, shape, 0) + i * RB
    return row < N


def _tc_a_body(x_ref, d0_ref, d1_ref, w1_ref, dinv_ref, g1_ref):
    dinv = lax.rsqrt(d0_ref[...] + d1_ref[...] + 1.0)
    h = jnp.dot(x_ref[...], w1_ref[...], preferred_element_type=jnp.float32)
    dinv_ref[...] = dinv
    g1_ref[...] = jnp.where(_row_mask((RB, F)), h * dinv, 0.0)


def _tc_b_body(a0_ref, a1_ref, g_ref, dinv_ref, b_ref, w2_ref, g2_ref):
    dinv = dinv_ref[...]
    h = dinv * (a0_ref[...] + a1_ref[...] + g_ref[...]) + b_ref[...]
    h = jnp.maximum(h, 0.0)
    h2 = jnp.dot(h, w2_ref[...], preferred_element_type=jnp.float32)
    g2_ref[...] = jnp.where(_row_mask((RB, F)), h2 * dinv, 0.0)


def _tc_c_body(a0_ref, a1_ref, g_ref, dinv_ref, b_ref,
               cw1_ref, cb1_ref, cw2_ref, cb2_ref, o1_ref, o2_ref):
    h = dinv_ref[...] * (a0_ref[...] + a1_ref[...] + g_ref[...]) + b_ref[...]
    h = jnp.maximum(h, 0.0)
    o1_ref[...] = jnp.dot(h, cw1_ref[...],
                          preferred_element_type=jnp.float32) + cb1_ref[...]
    o2_ref[...] = jnp.dot(h, cw2_ref[...],
                          preferred_element_type=jnp.float32) + cb2_ref[...]


def _rows(width):
    return pl.BlockSpec((RB, width), lambda i: (i, 0))


def _full(shape):
    return pl.BlockSpec(shape, lambda i: tuple(0 for _ in shape))


_tc_a = pl.pallas_call(
    _tc_a_body,
    grid=(GRID,),
    in_specs=[_rows(6), _rows(F), _rows(F), _full((6, F))],
    out_specs=[_rows(F), _rows(F)],
    out_shape=[jax.ShapeDtypeStruct((NPAD, F), jnp.float32),
               jax.ShapeDtypeStruct((NPAD, F), jnp.float32)],
)

_tc_b = pl.pallas_call(
    _tc_b_body,
    grid=(GRID,),
    in_specs=[_rows(F), _rows(F), _rows(F), _rows(F),
              _full((1, F)), _full((F, F))],
    out_specs=[_rows(F)],
    out_shape=[jax.ShapeDtypeStruct((NPAD, F), jnp.float32)],
)

_tc_c = pl.pallas_call(
    _tc_c_body,
    grid=(GRID,),
    in_specs=[_rows(F), _rows(F), _rows(F), _rows(F), _full((1, F)),
              _full((F, 13)), _full((1, 13)), _full((F, 8)), _full((1, 8))],
    out_specs=[_rows(13), _rows(8)],
    out_shape=[jax.ShapeDtypeStruct((NPAD, 13), jnp.float32),
               jax.ShapeDtypeStruct((NPAD, 8), jnp.float32)],
)


@jax.jit
def kernel(x, edge_index, W1, b1, W2, b2, CW1, Cb1, CW2, Cb2):
    ei = edge_index.astype(jnp.int32)
    pad = jnp.full((2, E_PAD - E), N, jnp.int32)
    eip = jnp.concatenate([ei, pad], axis=1).reshape(2, NW, K, C)
    src_i = eip[0]
    dst_i = eip[1]
    xp = jnp.zeros((NPAD, 6), jnp.float32).at[:N].set(x)
    zeros16 = jnp.zeros((NPAD, F), jnp.float32)

    deg = _sc_deg(dst_i, zeros16)
    dinv, g1 = _tc_a(xp, deg[0], deg[1], W1)
    agg1 = _sc_agg(g1, src_i, dst_i, zeros16)
    (g2,) = _tc_b(agg1[0], agg1[1], g1, dinv, b1.reshape(1, F), W2)
    agg2 = _sc_agg(g2, src_i, dst_i, zeros16)
    o1, o2 = _tc_c(agg2[0], agg2[1], g2, dinv, b2.reshape(1, F),
                   CW1, Cb1.reshape(1, 13), CW2, Cb2.reshape(1, 8))
    return o1[:N], o2[:N]


# speedup vs baseline: 55.0475x; 1.0840x over previous
import jax
import jax.numpy as jnp
from jax import lax
from jax.experimental import pallas as pl
from jax.experimental.pallas import tpu as pltpu
from jax.experimental.pallas import tpu_sc as plsc

N = 100000
E = 3200000
F = 16
NC, NS = 2, 16
NW = NC * NS
C = 128
K = 784
S = 4
NSB = K // S
E_PAD = NW * K * C
RPT = 6400
NPAD = NS * RPT
RB = 4096
GRID = NPAD // RB
SD = 8
NSBD = K // SD


def _sc_mesh():
    return plsc.VectorSubcoreMesh(core_axis_name="c", subcore_axis_name="s")


def _deg_body(dst_i, zeros_hbm, out_hbm, acc_sh, ones_v, dst_v, isem, ssem):
    c = lax.axis_index("c")
    s = lax.axis_index("s")
    wid = c * NS + s
    r0 = s * RPT

    def fill(i, carry):
        ones_v[i] = jnp.ones((F,), jnp.float32)
        return carry
    lax.fori_loop(0, C, fill, 0)
    pltpu.sync_copy(zeros_hbm.at[pl.ds(r0, RPT)], acc_sh.at[pl.ds(r0, RPT)])
    plsc.subcore_barrier()

    def stage(sb, slot):
        pltpu.async_copy(dst_i.at[wid, pl.ds(sb * SD, SD)], dst_v.at[slot],
                         isem.at[slot])

    def wait_stage(sb, slot):
        pltpu.make_async_copy(dst_i.at[wid, pl.ds(sb * SD, SD)],
                              dst_v.at[slot], isem.at[slot]).wait()

    stage(0, 0)

    def sb_body(sb, carry):
        slot = lax.rem(sb, 2)
        nslot = 1 - slot

        @pl.when(sb >= 1)
        def _():
            def w(j, cc):
                pltpu.make_async_copy(ones_v, acc_sh.at[dst_v.at[nslot, j]],
                                      ssem.at[nslot, j]).wait()
                return cc
            lax.fori_loop(0, SD, w, 0)

        @pl.when(sb + 1 < NSBD)
        def _():
            stage(sb + 1, nslot)

        wait_stage(sb, slot)

        def ch(j, cc):
            pltpu.async_copy(ones_v, acc_sh.at[dst_v.at[slot, j]],
                             ssem.at[slot, j], add=True)
            return cc
        lax.fori_loop(0, SD, ch, 0)
        return carry
    lax.fori_loop(0, NSBD, sb_body, 0)

    def dr(j, cc):
        pltpu.make_async_copy(ones_v, acc_sh.at[dst_v.at[(NSBD - 1) % 2, j]],
                              ssem.at[(NSBD - 1) % 2, j]).wait()
        return cc
    lax.fori_loop(0, SD, dr, 0)
    plsc.subcore_barrier()
    pltpu.sync_copy(acc_sh.at[pl.ds(r0, RPT)], out_hbm.at[c, pl.ds(r0, RPT)])


_sc_deg = pl.kernel(
    _deg_body,
    out_type=jax.ShapeDtypeStruct((NC, NPAD, F), jnp.float32),
    mesh=_sc_mesh(),
    compiler_params=pltpu.CompilerParams(use_tc_tiling_on_sc=False),
    scratch_types=[
        pltpu.VMEM_SHARED((NPAD, F), jnp.float32),
        pltpu.VMEM((C, F), jnp.float32),
        pltpu.VMEM((2, SD, C), jnp.int32),
        pltpu.SemaphoreType.DMA((2,)),
        pltpu.SemaphoreType.DMA((2, SD)),
    ],
)


def _agg_body(g_hbm, src_i, dst_i, zeros_hbm, out_hbm,
              acc_sh, src_v, dst_v, rows_v, isem, gsem, ssem):
    c = lax.axis_index("c")
    s = lax.axis_index("s")
    wid = c * NS + s
    r0 = s * RPT
    pltpu.sync_copy(zeros_hbm.at[pl.ds(r0, RPT)], acc_sh.at[pl.ds(r0, RPT)])
    plsc.subcore_barrier()

    def stage(sb, slot):
        pltpu.async_copy(src_i.at[wid, pl.ds(sb * S, S)], src_v.at[slot],
                         isem.at[slot, 0])
        pltpu.async_copy(dst_i.at[wid, pl.ds(sb * S, S)], dst_v.at[slot],
                         isem.at[slot, 1])

    def wait_stage(sb, slot):
        pltpu.make_async_copy(src_i.at[wid, pl.ds(sb * S, S)],
                              src_v.at[slot], isem.at[slot, 0]).wait()
        pltpu.make_async_copy(dst_i.at[wid, pl.ds(sb * S, S)],
                              dst_v.at[slot], isem.at[slot, 1]).wait()

    def fire(slot):
        def f(j, carry):
            pltpu.async_copy(g_hbm.at[src_v.at[slot, j]], rows_v.at[slot, j],
                             gsem.at[slot, j])
            return carry
        lax.fori_loop(0, S, f, 0)

    def drain(slot):
        def f(j, carry):
            pltpu.make_async_copy(g_hbm.at[src_v.at[slot, j]],
                                  rows_v.at[slot, j], gsem.at[slot, j]).wait()
            pltpu.async_copy(rows_v.at[slot, j], acc_sh.at[dst_v.at[slot, j]],
                             ssem.at[slot, j], add=True)
            return carry
        lax.fori_loop(0, S, f, 0)

    stage(0, 0)
    wait_stage(0, 0)
    fire(0)

    def sb_body(sb, carry):
        slot = lax.rem(sb, 2)
        nslot = 1 - slot

        @pl.when(sb >= 1)
        def _():
            def w(j, cc):
                pltpu.make_async_copy(rows_v.at[nslot, j],
                                      acc_sh.at[dst_v.at[nslot, j]],
                                      ssem.at[nslot, j]).wait()
                return cc
            lax.fori_loop(0, S, w, 0)

        @pl.when(sb + 1 < NSB)
        def _():
            stage(sb + 1, nslot)

        drain(slot)

        @pl.when(sb + 1 < NSB)
        def _():
            wait_stage(sb + 1, nslot)
            fire(nslot)
        return carry
    lax.fori_loop(0, NSB, sb_body, 0)

    def dr(j, cc):
        pltpu.make_async_copy(rows_v.at[(NSB - 1) % 2, j],
                              acc_sh.at[dst_v.at[(NSB - 1) % 2, j]],
                              ssem.at[(NSB - 1) % 2, j]).wait()
        return cc
    lax.fori_loop(0, S, dr, 0)
    plsc.subcore_barrier()
    pltpu.sync_copy(acc_sh.at[pl.ds(r0, RPT)], out_hbm.at[c, pl.ds(r0, RPT)])


_sc_agg = pl.kernel(
    _agg_body,
    out_type=jax.ShapeDtypeStruct((NC, NPAD, F), jnp.float32),
    mesh=_sc_mesh(),
    compiler_params=pltpu.CompilerParams(use_tc_tiling_on_sc=False),
    scratch_types=[
        pltpu.VMEM_SHARED((NPAD, F), jnp.float32),
        pltpu.VMEM((2, S, C), jnp.int32),
        pltpu.VMEM((2, S, C), jnp.int32),
        pltpu.VMEM((2, S, C, F), jnp.float32),
        pltpu.SemaphoreType.DMA((2, 2)),
        pltpu.SemaphoreType.DMA((2, S)),
        pltpu.SemaphoreType.DMA((2, S)),
    ],
)


def _row_mask(shape):
    i = pl.program_id(0)
    row = lax.broadcasted_iota(jnp.int32, shape, 0) + i * RB
    return row < N


def _tc_a_body(x_ref, d0_ref, d1_ref, w1_ref, dinv_ref, g1_ref):
    dinv = lax.rsqrt(d0_ref[...] + d1_ref[...] + 1.0)
    h = jnp.dot(x_ref[...], w1_ref[...], preferred_element_type=jnp.float32)
    dinv_ref[...] = dinv
    g1_ref[...] = jnp.where(_row_mask((RB, F)), h * dinv, 0.0)


def _tc_b_body(a0_ref, a1_ref, g_ref, dinv_ref, b_ref, w2_ref, g2_ref):
    dinv = dinv_ref[...]
    h = dinv * (a0_ref[...] + a1_ref[...] + g_ref[...]) + b_ref[...]
    h = jnp.maximum(h, 0.0)
    h2 = jnp.dot(h, w2_ref[...], preferred_element_type=jnp.float32)
    g2_ref[...] = jnp.where(_row_mask((RB, F)), h2 * dinv, 0.0)


def _tc_c_body(a0_ref, a1_ref, g_ref, dinv_ref, b_ref,
               cw1_ref, cb1_ref, cw2_ref, cb2_ref, o1_ref, o2_ref):
    h = dinv_ref[...] * (a0_ref[...] + a1_ref[...] + g_ref[...]) + b_ref[...]
    h = jnp.maximum(h, 0.0)
    o1_ref[...] = jnp.dot(h, cw1_ref[...],
                          preferred_element_type=jnp.float32) + cb1_ref[...]
    o2_ref[...] = jnp.dot(h, cw2_ref[...],
                          preferred_element_type=jnp.float32) + cb2_ref[...]


def _rows(width):
    return pl.BlockSpec((RB, width), lambda i: (i, 0))


def _full(shape):
    return pl.BlockSpec(shape, lambda i: tuple(0 for _ in shape))


_tc_a = pl.pallas_call(
    _tc_a_body,
    grid=(GRID,),
    in_specs=[_rows(6), _rows(F), _rows(F), _full((6, F))],
    out_specs=[_rows(F), _rows(F)],
    out_shape=[jax.ShapeDtypeStruct((NPAD, F), jnp.float32),
               jax.ShapeDtypeStruct((NPAD, F), jnp.float32)],
)

_tc_b = pl.pallas_call(
    _tc_b_body,
    grid=(GRID,),
    in_specs=[_rows(F), _rows(F), _rows(F), _rows(F),
              _full((1, F)), _full((F, F))],
    out_specs=[_rows(F)],
    out_shape=[jax.ShapeDtypeStruct((NPAD, F), jnp.float32)],
)

_tc_c = pl.pallas_call(
    _tc_c_body,
    grid=(GRID,),
    in_specs=[_rows(F), _rows(F), _rows(F), _rows(F), _full((1, F)),
              _full((F, 13)), _full((1, 13)), _full((F, 8)), _full((1, 8))],
    out_specs=[_rows(13), _rows(8)],
    out_shape=[jax.ShapeDtypeStruct((N, 13), jnp.float32),
               jax.ShapeDtypeStruct((N, 8), jnp.float32)],
)


@jax.jit
def kernel(x, edge_index, W1, b1, W2, b2, CW1, Cb1, CW2, Cb2):
    ei = edge_index.astype(jnp.int32)
    pad = jnp.full((2, E_PAD - E), N, jnp.int32)
    eip = jnp.concatenate([ei, pad], axis=1).reshape(2, NW, K, C)
    src_i = eip[0]
    dst_i = eip[1]
    xp = jnp.zeros((NPAD, 6), jnp.float32).at[:N].set(x)
    zeros16 = jnp.zeros((NPAD, F), jnp.float32)

    deg = _sc_deg(dst_i, zeros16)
    dinv, g1 = _tc_a(xp, deg[0], deg[1], W1)
    agg1 = _sc_agg(g1, src_i, dst_i, zeros16)
    (g2,) = _tc_b(agg1[0], agg1[1], g1, dinv, b1.reshape(1, F), W2)
    agg2 = _sc_agg(g2, src_i, dst_i, zeros16)
    o1, o2 = _tc_c(agg2[0], agg2[1], g2, dinv, b2.reshape(1, F),
                   CW1, Cb1.reshape(1, 13), CW2, Cb2.reshape(1, 8))
    return o1, o2


# speedup vs baseline: 60.6072x; 1.1010x over previous
import jax
import jax.numpy as jnp
from jax import lax
from jax.experimental import pallas as pl
from jax.experimental.pallas import tpu as pltpu
from jax.experimental.pallas import tpu_sc as plsc

N = 100000
E = 3200000
F = 16
NC, NS = 2, 16
NW = NC * NS
C = 128
NCH = E // C
CW_LO = NCH // NW
NEXTRA = NCH % NW
S = 4
NSB = -(-(CW_LO + 1) // S)
SD = 8
NSBD = -(-(CW_LO + 1) // SD)
RPT = 6400
NPAD = NS * RPT
RB = 4096
GRID = NPAD // RB


def _sc_mesh():
    return plsc.VectorSubcoreMesh(core_axis_name="c", subcore_axis_name="s")


def _chunk_range(wid):
    start = wid * CW_LO + jnp.minimum(wid, NEXTRA)
    cnt = CW_LO + jnp.where(wid < NEXTRA, 1, 0)
    return start, cnt


def _deg_body(dst_i, zeros_hbm, out_hbm, acc_sh, ones_v, dst_v, isem, ssem):
    c = lax.axis_index("c")
    s = lax.axis_index("s")
    wid = c * NS + s
    r0 = s * RPT
    start, cnt = _chunk_range(wid)

    def off(sb):
        return jnp.minimum(start + sb * SD, NCH - SD)

    def pred(sb, j):
        gc = off(sb) + j
        return (gc >= start + sb * SD) & (gc < start + cnt)

    def fill(i, carry):
        ones_v[i] = jnp.ones((F,), jnp.float32)
        return carry
    lax.fori_loop(0, C, fill, 0)
    pltpu.sync_copy(zeros_hbm.at[pl.ds(r0, RPT)], acc_sh.at[pl.ds(r0, RPT)])
    plsc.subcore_barrier()

    def stage(sb, slot):
        pltpu.async_copy(dst_i.at[pl.ds(off(sb), SD)], dst_v.at[slot],
                         isem.at[slot])

    def wait_stage(sb, slot):
        pltpu.make_async_copy(dst_i.at[pl.ds(off(sb), SD)],
                              dst_v.at[slot], isem.at[slot]).wait()

    stage(0, 0)

    def sb_body(sb, carry):
        slot = lax.rem(sb, 2)
        nslot = 1 - slot

        @pl.when(sb >= 1)
        def _():
            def w(j, cc):
                @pl.when(pred(sb - 1, j))
                def _():
                    pltpu.make_async_copy(ones_v,
                                          acc_sh.at[dst_v.at[nslot, j]],
                                          ssem.at[nslot, j]).wait()
                return cc
            lax.fori_loop(0, SD, w, 0)

        @pl.when(sb + 1 < NSBD)
        def _():
            stage(sb + 1, nslot)

        wait_stage(sb, slot)

        def ch(j, cc):
            @pl.when(pred(sb, j))
            def _():
                pltpu.async_copy(ones_v, acc_sh.at[dst_v.at[slot, j]],
                                 ssem.at[slot, j], add=True)
            return cc
        lax.fori_loop(0, SD, ch, 0)
        return carry
    lax.fori_loop(0, NSBD, sb_body, 0)

    def dr(j, cc):
        @pl.when(pred(NSBD - 1, j))
        def _():
            pltpu.make_async_copy(ones_v,
                                  acc_sh.at[dst_v.at[(NSBD - 1) % 2, j]],
                                  ssem.at[(NSBD - 1) % 2, j]).wait()
        return cc
    lax.fori_loop(0, SD, dr, 0)
    plsc.subcore_barrier()
    pltpu.sync_copy(acc_sh.at[pl.ds(r0, RPT)], out_hbm.at[c, pl.ds(r0, RPT)])


_sc_deg = pl.kernel(
    _deg_body,
    out_type=jax.ShapeDtypeStruct((NC, NPAD, F), jnp.float32),
    mesh=_sc_mesh(),
    compiler_params=pltpu.CompilerParams(use_tc_tiling_on_sc=False),
    scratch_types=[
        pltpu.VMEM_SHARED((NPAD, F), jnp.float32),
        pltpu.VMEM((C, F), jnp.float32),
        pltpu.VMEM((2, SD, C), jnp.int32),
        pltpu.SemaphoreType.DMA((2,)),
        pltpu.SemaphoreType.DMA((2, SD)),
    ],
)


def _agg_body(g_hbm, src_i, dst_i, zeros_hbm, out_hbm,
              acc_sh, src_v, dst_v, rows_v, isem, gsem, ssem):
    c = lax.axis_index("c")
    s = lax.axis_index("s")
    wid = c * NS + s
    r0 = s * RPT
    start, cnt = _chunk_range(wid)

    def off(sb):
        return jnp.minimum(start + sb * S, NCH - S)

    def pred(sb, j):
        gc = off(sb) + j
        return (gc >= start + sb * S) & (gc < start + cnt)

    pltpu.sync_copy(zeros_hbm.at[pl.ds(r0, RPT)], acc_sh.at[pl.ds(r0, RPT)])
    plsc.subcore_barrier()

    def stage(sb, slot):
        pltpu.async_copy(src_i.at[pl.ds(off(sb), S)], src_v.at[slot],
                         isem.at[slot, 0])
        pltpu.async_copy(dst_i.at[pl.ds(off(sb), S)], dst_v.at[slot],
                         isem.at[slot, 1])

    def wait_stage(sb, slot):
        pltpu.make_async_copy(src_i.at[pl.ds(off(sb), S)],
                              src_v.at[slot], isem.at[slot, 0]).wait()
        pltpu.make_async_copy(dst_i.at[pl.ds(off(sb), S)],
                              dst_v.at[slot], isem.at[slot, 1]).wait()

    def fire(sb, slot):
        def f(j, carry):
            @pl.when(pred(sb, j))
            def _():
                pltpu.async_copy(g_hbm.at[src_v.at[slot, j]],
                                 rows_v.at[slot, j], gsem.at[slot, j])
            return carry
        lax.fori_loop(0, S, f, 0)

    def drain(sb, slot):
        def f(j, carry):
            @pl.when(pred(sb, j))
            def _():
                pltpu.make_async_copy(g_hbm.at[src_v.at[slot, j]],
                                      rows_v.at[slot, j],
                                      gsem.at[slot, j]).wait()
                pltpu.async_copy(rows_v.at[slot, j],
                                 acc_sh.at[dst_v.at[slot, j]],
                                 ssem.at[slot, j], add=True)
            return carry
        lax.fori_loop(0, S, f, 0)

    stage(0, 0)
    wait_stage(0, 0)
    fire(0, 0)

    def sb_body(sb, carry):
        slot = lax.rem(sb, 2)
        nslot = 1 - slot

        @pl.when(sb >= 1)
        def _():
            def w(j, cc):
                @pl.when(pred(sb - 1, j))
                def _():
                    pltpu.make_async_copy(rows_v.at[nslot, j],
                                          acc_sh.at[dst_v.at[nslot, j]],
                                          ssem.at[nslot, j]).wait()
                return cc
            lax.fori_loop(0, S, w, 0)

        @pl.when(sb + 1 < NSB)
        def _():
            stage(sb + 1, nslot)

        drain(sb, slot)

        @pl.when(sb + 1 < NSB)
        def _():
            wait_stage(sb + 1, nslot)
            fire(sb + 1, nslot)
        return carry
    lax.fori_loop(0, NSB, sb_body, 0)

    def dr(j, cc):
        @pl.when(pred(NSB - 1, j))
        def _():
            pltpu.make_async_copy(rows_v.at[(NSB - 1) % 2, j],
                                  acc_sh.at[dst_v.at[(NSB - 1) % 2, j]],
                                  ssem.at[(NSB - 1) % 2, j]).wait()
        return cc
    lax.fori_loop(0, S, dr, 0)
    plsc.subcore_barrier()
    pltpu.sync_copy(acc_sh.at[pl.ds(r0, RPT)], out_hbm.at[c, pl.ds(r0, RPT)])


_sc_agg = pl.kernel(
    _agg_body,
    out_type=jax.ShapeDtypeStruct((NC, NPAD, F), jnp.float32),
    mesh=_sc_mesh(),
    compiler_params=pltpu.CompilerParams(use_tc_tiling_on_sc=False),
    scratch_types=[
        pltpu.VMEM_SHARED((NPAD, F), jnp.float32),
        pltpu.VMEM((2, S, C), jnp.int32),
        pltpu.VMEM((2, S, C), jnp.int32),
        pltpu.VMEM((2, S, C, F), jnp.float32),
        pltpu.SemaphoreType.DMA((2, 2)),
        pltpu.SemaphoreType.DMA((2, S)),
        pltpu.SemaphoreType.DMA((2, S)),
    ],
)


def _tc_a_body(x_ref, d0_ref, d1_ref, w1_ref, dinv_ref, g1_ref):
    dinv = lax.rsqrt(d0_ref[...] + d1_ref[...] + 1.0)
    h = jnp.dot(x_ref[...], w1_ref[...], preferred_element_type=jnp.float32)
    dinv_ref[...] = dinv
    g1_ref[...] = h * dinv


def _tc_b_body(a0_ref, a1_ref, g_ref, dinv_ref, b_ref, w2_ref, g2_ref):
    dinv = dinv_ref[...]
    h = dinv * (a0_ref[...] + a1_ref[...] + g_ref[...]) + b_ref[...]
    h = jnp.maximum(h, 0.0)
    h2 = jnp.dot(h, w2_ref[...], preferred_element_type=jnp.float32)
    g2_ref[...] = h2 * dinv


def _tc_c_body(a0_ref, a1_ref, g_ref, dinv_ref, b_ref,
               cw1_ref, cb1_ref, cw2_ref, cb2_ref, o1_ref, o2_ref):
    h = dinv_ref[...] * (a0_ref[...] + a1_ref[...] + g_ref[...]) + b_ref[...]
    h = jnp.maximum(h, 0.0)
    o1_ref[...] = jnp.dot(h, cw1_ref[...],
                          preferred_element_type=jnp.float32) + cb1_ref[...]
    o2_ref[...] = jnp.dot(h, cw2_ref[...],
                          preferred_element_type=jnp.float32) + cb2_ref[...]


def _rows(width):
    return pl.BlockSpec((RB, width), lambda i: (i, 0))


def _full(shape):
    return pl.BlockSpec(shape, lambda i: tuple(0 for _ in shape))


_tc_a = pl.pallas_call(
    _tc_a_body,
    grid=(GRID,),
    in_specs=[_rows(6), _rows(F), _rows(F), _full((6, F))],
    out_specs=[_rows(F), _rows(F)],
    out_shape=[jax.ShapeDtypeStruct((NPAD, F), jnp.float32),
               jax.ShapeDtypeStruct((NPAD, F), jnp.float32)],
)

_tc_b = pl.pallas_call(
    _tc_b_body,
    grid=(GRID,),
    in_specs=[_rows(F), _rows(F), _rows(F), _rows(F),
              _full((1, F)), _full((F, F))],
    out_specs=[_rows(F)],
    out_shape=[jax.ShapeDtypeStruct((NPAD, F), jnp.float32)],
)

_tc_c = pl.pallas_call(
    _tc_c_body,
    grid=(GRID,),
    in_specs=[_rows(F), _rows(F), _rows(F), _rows(F), _full((1, F)),
              _full((F, 13)), _full((1, 13)), _full((F, 8)), _full((1, 8))],
    out_specs=[_rows(13), _rows(8)],
    out_shape=[jax.ShapeDtypeStruct((N, 13), jnp.float32),
               jax.ShapeDtypeStruct((N, 8), jnp.float32)],
)


@jax.jit
def kernel(x, edge_index, W1, b1, W2, b2, CW1, Cb1, CW2, Cb2):
    ei = edge_index.astype(jnp.int32).reshape(2, NCH, C)
    src_i = ei[0]
    dst_i = ei[1]
    xp = jnp.zeros((NPAD, 6), jnp.float32).at[:N].set(x)
    zeros16 = jnp.zeros((NPAD, F), jnp.float32)

    deg = _sc_deg(dst_i, zeros16)
    dinv, g1 = _tc_a(xp, deg[0], deg[1], W1)
    agg1 = _sc_agg(g1, src_i, dst_i, zeros16)
    (g2,) = _tc_b(agg1[0], agg1[1], g1, dinv, b1.reshape(1, F), W2)
    agg2 = _sc_agg(g2, src_i, dst_i, zeros16)
    o1, o2 = _tc_c(agg2[0], agg2[1], g2, dinv, b2.reshape(1, F),
                   CW1, Cb1.reshape(1, 13), CW2, Cb2.reshape(1, 8))
    return o1, o2
